# Initial kernel scaffold; baseline (speedup 1.0000x reference)
import jax
import jax.numpy as jnp
from jax.experimental import pallas as pl


def kernel(bank, bank_labels, k, labels, index, logits_x_ulb, p_model, label_hist, time_p):
    raise NotImplementedError("write your pallas kernel here")


# speedup vs baseline: 1.4761x; 1.4761x over previous
import functools

import jax
import jax.numpy as jnp
from jax import lax
from jax.experimental import pallas as pl
from jax.experimental.pallas import tpu as pltpu

M_EMA = 0.999
B = 16384
D = 128
K = 100000
C = 1000

BS1 = 512
NB1 = B // BS1
BS2 = 2048
NB2 = B // BS2


def _pass1_body(logits_ref, p_model_ref, label_hist_ref, time_p_ref,
                pseudo_ref, maxp_ref, idx_ref,
                p_model_out, label_hist_out, time_p_out, thr_out,
                psum, hsum, msum):
    i = pl.program_id(0)

    @pl.when(i == 0)
    def _init():
        psum[...] = jnp.zeros_like(psum)
        hsum[...] = jnp.zeros_like(hsum)
        msum[0] = 0.0

    l = logits_ref[...]
    m = jnp.max(l, axis=1, keepdims=True)
    e = jnp.exp(l - m)
    z = jnp.sum(e, axis=1, keepdims=True)
    probs = e / z
    maxp = 1.0 / z[:, 0]
    idx = jnp.argmax(l, axis=1).astype(jnp.int32)
    onehot = (lax.broadcasted_iota(jnp.int32, (BS1, C), 1) == idx[:, None])
    onehot_f = onehot.astype(jnp.float32)

    pseudo_ref[...] = onehot_f
    maxp_ref[...] = maxp
    idx_ref[...] = idx

    psum[...] = psum[...] + jnp.sum(probs, axis=0)
    hsum[...] = hsum[...] + jnp.sum(onehot_f, axis=0)
    msum[0] = msum[0] + jnp.sum(maxp)

    @pl.when(i == NB1 - 1)
    def _fin():
        inv_b = 1.0 / B
        new_p = p_model_ref[...] * M_EMA + (1.0 - M_EMA) * (psum[...] * inv_b)
        new_h = label_hist_ref[...] * M_EMA + (1.0 - M_EMA) * (hsum[...] * inv_b)
        new_t = time_p_ref[0] * M_EMA + (1.0 - M_EMA) * (msum[0] * inv_b)
        p_model_out[...] = new_p
        label_hist_out[...] = new_h
        time_p_out[0] = new_t
        thr_out[...] = new_t * (new_p / jnp.max(new_p))


def _pass2_body(maxp_ref, idx_ref, thr_ref, mask_ref):
    idx = idx_ref[...]
    thr = thr_ref[...]
    oh = (lax.broadcasted_iota(jnp.int32, (BS2, C), 1) == idx[:, None])
    t = jnp.sum(jnp.where(oh, thr[None, :], 0.0), axis=1)
    mask_ref[...] = (maxp_ref[...] >= t).astype(jnp.float32)


@functools.partial(jax.jit, static_argnames=())
def _dense(logits, p_model, label_hist, time_p):
    out_shapes = (
        jax.ShapeDtypeStruct((B, C), jnp.float32),
        jax.ShapeDtypeStruct((B,), jnp.float32),
        jax.ShapeDtypeStruct((B,), jnp.int32),
        jax.ShapeDtypeStruct((C,), jnp.float32),
        jax.ShapeDtypeStruct((C,), jnp.float32),
        jax.ShapeDtypeStruct((1,), jnp.float32),
        jax.ShapeDtypeStruct((C,), jnp.float32),
    )
    pseudo, maxp, idx, new_p, new_h, new_t, thr = pl.pallas_call(
        _pass1_body,
        grid=(NB1,),
        in_specs=[
            pl.BlockSpec((BS1, C), lambda i: (i, 0)),
            pl.BlockSpec((C,), lambda i: (0,)),
            pl.BlockSpec((C,), lambda i: (0,)),
            pl.BlockSpec(memory_space=pltpu.SMEM),
        ],
        out_specs=(
            pl.BlockSpec((BS1, C), lambda i: (i, 0)),
            pl.BlockSpec((BS1,), lambda i: (i,)),
            pl.BlockSpec((BS1,), lambda i: (i,)),
            pl.BlockSpec((C,), lambda i: (0,)),
            pl.BlockSpec((C,), lambda i: (0,)),
            pl.BlockSpec(memory_space=pltpu.SMEM),
            pl.BlockSpec((C,), lambda i: (0,)),
        ),
        scratch_shapes=[
            pltpu.VMEM((C,), jnp.float32),
            pltpu.VMEM((C,), jnp.float32),
            pltpu.SMEM((1,), jnp.float32),
        ],
        out_shape=out_shapes,
    )(logits, p_model, label_hist, time_p)

    mask = pl.pallas_call(
        _pass2_body,
        grid=(NB2,),
        in_specs=[
            pl.BlockSpec((BS2,), lambda i: (i,)),
            pl.BlockSpec((BS2,), lambda i: (i,)),
            pl.BlockSpec((C,), lambda i: (0,)),
        ],
        out_specs=pl.BlockSpec((BS2,), lambda i: (i,)),
        out_shape=jax.ShapeDtypeStruct((B,), jnp.float32),
    )(maxp, idx, thr)

    return pseudo, mask, new_p, new_h, new_t


def kernel(bank, bank_labels, k, labels, index, logits_x_ulb, p_model,
           label_hist, time_p):
    new_bank = bank.at[:, index].set(k.T)
    new_bank_labels = bank_labels.at[index].set(labels)

    pseudo, mask, new_p, new_h, new_t = _dense(
        logits_x_ulb, p_model, label_hist, time_p)

    return (new_bank, new_bank_labels, mask, pseudo, new_p, new_h, new_t)
